# Initial kernel scaffold; baseline (speedup 1.0000x reference)
import jax
import jax.numpy as jnp
from jax.experimental import pallas as pl


def kernel(hidden_states, top_k_index, top_k_weights, gate_up_proj, down_proj):
    raise NotImplementedError("write your pallas kernel here")


# speedup vs baseline: 1.5411x; 1.5411x over previous
import functools

import jax
import jax.numpy as jnp
from jax import lax
from jax.experimental import pallas as pl
from jax.experimental.pallas import tpu as pltpu
from jax.experimental.pallas import tpu_sc as plsc

_E = 16
_K = 2
_H = 1024
_I = 512
_T = 4096
_B = 128
_P = 10240
_NB = _P // _B
_NC = 2
_NS = 16
_NW = _NC * _NS
_GC = 64
_CC = 32

_MESH = dict(core_axis_name="c", subcore_axis_name="s")


@functools.partial(
    pl.kernel,
    out_type=jax.ShapeDtypeStruct((_P, _H), jnp.float32),
    mesh=plsc.VectorSubcoreMesh(**_MESH),
    scratch_types=[
        pltpu.VMEM((_GC,), jnp.int32),
        pltpu.VMEM((_GC, _H), jnp.float32),
        pltpu.SemaphoreType.DMA,
    ],
)
def _gather_rows(src, idx, out, idx_v, rows_v, sem):
    wid = lax.axis_index("s") * _NC + lax.axis_index("c")
    base = wid * (_P // _NW)
    for it in range(_P // _NW // _GC):
        o = base + it * _GC
        pltpu.sync_copy(idx.at[pl.ds(o, _GC)], idx_v)
        pltpu.async_copy(src.at[idx_v], rows_v, sem).wait()
        pltpu.sync_copy(rows_v, out.at[pl.ds(o, _GC)])


@functools.partial(
    pl.kernel,
    out_type=jax.ShapeDtypeStruct((_T, _H), jnp.float32),
    mesh=plsc.VectorSubcoreMesh(**_MESH),
    scratch_types=[
        pltpu.VMEM((_CC,), jnp.int32),
        pltpu.VMEM((_CC,), jnp.int32),
        pltpu.VMEM((_CC, _H), jnp.float32),
        pltpu.VMEM((_CC, _H), jnp.float32),
        pltpu.SemaphoreType.DMA,
    ],
)
def _combine_rows(y, pos0, pos1, out, i0_v, i1_v, a_v, b_v, sem):
    wid = lax.axis_index("s") * _NC + lax.axis_index("c")
    base = wid * (_T // _NW)
    for it in range(_T // _NW // _CC):
        o = base + it * _CC
        pltpu.sync_copy(pos0.at[pl.ds(o, _CC)], i0_v)
        pltpu.sync_copy(pos1.at[pl.ds(o, _CC)], i1_v)
        pltpu.async_copy(y.at[i0_v], a_v, sem).wait()
        pltpu.async_copy(y.at[i1_v], b_v, sem).wait()

        def add_body(n, _):
            r = n // (_H // 16)
            c = (n % (_H // 16)) * 16
            a_v[r, pl.ds(c, 16)] = a_v[r, pl.ds(c, 16)] + b_v[r, pl.ds(c, 16)]
            return 0

        lax.fori_loop(0, _CC * (_H // 16), add_body, 0)
        pltpu.sync_copy(a_v, out.at[pl.ds(o, _CC)])


def _ffn_block(be_ref, x_ref, gu_ref, dp_ref, w_ref, y_ref):
    x = x_ref[...]
    gu = gu_ref[0]
    g = lax.dot_general(x, gu, (((1,), (1,)), ((), ())),
                        preferred_element_type=jnp.float32)
    gate = g[:, :_I]
    up = g[:, _I:]
    h = up * (gate * jax.nn.sigmoid(gate))
    dp = dp_ref[0]
    y = lax.dot_general(h, dp, (((1,), (1,)), ((), ())),
                        preferred_element_type=jnp.float32)
    y_ref[...] = y * w_ref[...]


_ffn = pl.pallas_call(
    _ffn_block,
    grid_spec=pltpu.PrefetchScalarGridSpec(
        num_scalar_prefetch=1,
        grid=(_NB,),
        in_specs=[
            pl.BlockSpec((_B, _H), lambda b, be: (b, 0)),
            pl.BlockSpec((1, 2 * _I, _H), lambda b, be: (be[b], 0, 0)),
            pl.BlockSpec((1, _H, _I), lambda b, be: (be[b], 0, 0)),
            pl.BlockSpec((_B, 1), lambda b, be: (b, 0)),
        ],
        out_specs=pl.BlockSpec((_B, _H), lambda b, be: (b, 0)),
    ),
    out_shape=jax.ShapeDtypeStruct((_P, _H), jnp.float32),
    compiler_params=pltpu.CompilerParams(
        dimension_semantics=("arbitrary",),
    ),
)


def kernel(hidden_states, top_k_index, top_k_weights, gate_up_proj, down_proj):
    flat_e = top_k_index.astype(jnp.int32).reshape(-1)
    oh = (flat_e[:, None] == jnp.arange(_E, dtype=jnp.int32)[None, :])
    csum = jnp.cumsum(oh.astype(jnp.int32), axis=0)
    counts = csum[-1]
    rank = jnp.take_along_axis(csum, flat_e[:, None], axis=1)[:, 0] - 1
    padded = ((counts + _B - 1) // _B) * _B
    ends = jnp.cumsum(padded)
    offsets = ends - padded
    pos = offsets[flat_e] + rank
    tok = jnp.arange(_T * _K, dtype=jnp.int32) // _K
    rows_src = jnp.zeros((_P,), jnp.int32).at[pos].set(tok)
    w_sorted = jnp.zeros((_P,), jnp.float32).at[pos].set(
        top_k_weights.astype(jnp.float32).reshape(-1))
    block_expert = jnp.minimum(
        jnp.searchsorted(ends, jnp.arange(_NB, dtype=jnp.int32) * _B,
                         side="right"),
        _E - 1).astype(jnp.int32)
    pos_tk = pos.reshape(_T, _K)
    pos0 = pos_tk[:, 0]
    pos1 = pos_tk[:, 1]

    x_sorted = _gather_rows(hidden_states, rows_src)
    y_sorted = _ffn(block_expert, x_sorted, gate_up_proj, down_proj,
                    w_sorted[:, None])
    return _combine_rows(y_sorted, pos0, pos1)


# speedup vs baseline: 2.1833x; 1.4167x over previous
import functools

import jax
import jax.numpy as jnp
from jax import lax
from jax.experimental import pallas as pl
from jax.experimental.pallas import tpu as pltpu
from jax.experimental.pallas import tpu_sc as plsc

_E = 16
_K = 2
_H = 1024
_I = 512
_T = 4096
_B = 128
_P = 10240
_NB = _P // _B
_NC = 2
_NS = 16
_NW = _NC * _NS
_TW = _T // _NW
_GC = 32
_NGC = _TW // _GC
_CC = 16
_NCC = _TW // _CC
_V = 16

_MESH = dict(core_axis_name="c", subcore_axis_name="s")


@functools.partial(
    pl.kernel,
    out_type=jax.ShapeDtypeStruct((_P, _H), jnp.float32),
    mesh=plsc.VectorSubcoreMesh(**_MESH),
    scratch_types=[
        pltpu.VMEM((_GC,), jnp.int32),
        pltpu.VMEM((_GC,), jnp.int32),
        pltpu.VMEM((_GC,), jnp.int32),
        pltpu.VMEM((_GC,), jnp.int32),
        pltpu.VMEM((_GC, _H), jnp.float32),
        pltpu.VMEM((_GC, _H), jnp.float32),
        pltpu.SemaphoreType.DMA,
        pltpu.SemaphoreType.DMA,
    ],
)
def _permute_rows(src, pos0, pos1, out, i0a, i0b, i1a, i1b, bufa, bufb,
                  sem_in, sem_out):
    wid = lax.axis_index("s") * _NC + lax.axis_index("c")
    base = wid * _TW
    i0 = (i0a, i0b)
    i1 = (i1a, i1b)
    buf = (bufa, bufb)
    in_h = [None] * _NGC
    out_h = [None] * _NGC

    def load_idx(c):
        o = base + c * _GC
        pltpu.sync_copy(pos0.at[pl.ds(o, _GC)], i0[c % 2])
        pltpu.sync_copy(pos1.at[pl.ds(o, _GC)], i1[c % 2])

    load_idx(0)
    in_h[0] = pltpu.async_copy(src.at[pl.ds(base, _GC)], buf[0], sem_in)
    for c in range(_NGC):
        in_h[c].wait()
        if c + 1 < _NGC:
            if c >= 1:
                for h in out_h[c - 1]:
                    h.wait()
            load_idx(c + 1)
            o = base + (c + 1) * _GC
            in_h[c + 1] = pltpu.async_copy(
                src.at[pl.ds(o, _GC)], buf[(c + 1) % 2], sem_in)
        out_h[c] = (
            pltpu.async_copy(buf[c % 2], out.at[i0[c % 2]], sem_out),
            pltpu.async_copy(buf[c % 2], out.at[i1[c % 2]], sem_out),
        )
    for c in (_NGC - 2, _NGC - 1):
        for h in out_h[c]:
            h.wait()


@functools.partial(
    pl.kernel,
    out_type=jax.ShapeDtypeStruct((_T, _H), jnp.float32),
    mesh=plsc.VectorSubcoreMesh(**_MESH),
    scratch_types=[
        pltpu.VMEM((_CC,), jnp.int32),
        pltpu.VMEM((_CC,), jnp.int32),
        pltpu.VMEM((_CC,), jnp.int32),
        pltpu.VMEM((_CC,), jnp.int32),
        pltpu.VMEM((_CC, _V), jnp.float32),
        pltpu.VMEM((_CC, _V), jnp.float32),
        pltpu.VMEM((_CC, _V), jnp.float32),
        pltpu.VMEM((_CC, _V), jnp.float32),
        pltpu.VMEM((_CC, _H), jnp.float32),
        pltpu.VMEM((_CC, _H), jnp.float32),
        pltpu.VMEM((_CC, _H), jnp.float32),
        pltpu.VMEM((_CC, _H), jnp.float32),
        pltpu.SemaphoreType.DMA,
        pltpu.SemaphoreType.DMA,
    ],
)
def _combine_rows(y, pos0, pos1, w0m, w1m, out, i0a, i0b, i1a, i1b,
                  w0a, w0b, w1a, w1b, a0, a1, b0, b1, sem_in, sem_out):
    wid = lax.axis_index("s") * _NC + lax.axis_index("c")
    base = wid * _TW
    i0 = (i0a, i0b)
    i1 = (i1a, i1b)
    w0v = (w0a, w0b)
    w1v = (w1a, w1b)
    av = (a0, a1)
    bv = (b0, b1)
    in_h = [None] * _NCC
    out_h = [None] * _NCC

    def start_chunk(c):
        o = base + c * _CC
        pltpu.sync_copy(pos0.at[pl.ds(o, _CC)], i0[c % 2])
        pltpu.sync_copy(pos1.at[pl.ds(o, _CC)], i1[c % 2])
        pltpu.sync_copy(w0m.at[pl.ds(o, _CC)], w0v[c % 2])
        pltpu.sync_copy(w1m.at[pl.ds(o, _CC)], w1v[c % 2])
        in_h[c] = (
            pltpu.async_copy(y.at[i0[c % 2]], av[c % 2], sem_in),
            pltpu.async_copy(y.at[i1[c % 2]], bv[c % 2], sem_in),
        )

    start_chunk(0)
    for c in range(_NCC):
        for h in in_h[c]:
            h.wait()
        if c + 1 < _NCC:
            if c >= 1:
                out_h[c - 1].wait()
            start_chunk(c + 1)
        a_r, b_r = av[c % 2], bv[c % 2]
        w0_r, w1_r = w0v[c % 2], w1v[c % 2]

        def row_body(r, _):
            w0 = w0_r[r, pl.ds(0, _V)]
            w1 = w1_r[r, pl.ds(0, _V)]
            for j in range(_H // _V):
                s = pl.ds(j * _V, _V)
                a_r[r, s] = w0 * a_r[r, s] + w1 * b_r[r, s]
            return 0

        lax.fori_loop(0, _CC, row_body, 0)
        out_h[c] = pltpu.async_copy(
            a_r, out.at[pl.ds(base + c * _CC, _CC)], sem_out)
    out_h[_NCC - 2].wait()
    out_h[_NCC - 1].wait()


def _ffn_block(be_ref, x_ref, gu_ref, dp_ref, y_ref):
    x = x_ref[...]
    gu = gu_ref[0]
    g = lax.dot_general(x, gu, (((1,), (1,)), ((), ())),
                        preferred_element_type=jnp.float32)
    gate = g[:, :_I]
    up = g[:, _I:]
    h = up * (gate * jax.nn.sigmoid(gate))
    dp = dp_ref[0]
    y_ref[...] = lax.dot_general(h, dp, (((1,), (1,)), ((), ())),
                                 preferred_element_type=jnp.float32)


_ffn = pl.pallas_call(
    _ffn_block,
    grid_spec=pltpu.PrefetchScalarGridSpec(
        num_scalar_prefetch=1,
        grid=(_NB,),
        in_specs=[
            pl.BlockSpec((_B, _H), lambda b, be: (b, 0)),
            pl.BlockSpec((1, 2 * _I, _H), lambda b, be: (be[b], 0, 0)),
            pl.BlockSpec((1, _H, _I), lambda b, be: (be[b], 0, 0)),
        ],
        out_specs=pl.BlockSpec((_B, _H), lambda b, be: (b, 0)),
    ),
    out_shape=jax.ShapeDtypeStruct((_P, _H), jnp.float32),
    compiler_params=pltpu.CompilerParams(
        dimension_semantics=("arbitrary",),
    ),
)


def kernel(hidden_states, top_k_index, top_k_weights, gate_up_proj, down_proj):
    flat_e = top_k_index.astype(jnp.int32).reshape(-1)
    oh = (flat_e[:, None] == jnp.arange(_E, dtype=jnp.int32)[None, :])
    csum = jnp.cumsum(oh.astype(jnp.int32), axis=0)
    counts = csum[-1]
    rank = jnp.take_along_axis(csum, flat_e[:, None], axis=1)[:, 0] - 1
    padded = ((counts + _B - 1) // _B) * _B
    ends = jnp.cumsum(padded)
    offsets = ends - padded
    pos = offsets[flat_e] + rank
    block_expert = jnp.minimum(
        jnp.searchsorted(ends, jnp.arange(_NB, dtype=jnp.int32) * _B,
                         side="right"),
        _E - 1).astype(jnp.int32)
    pos_tk = pos.reshape(_T, _K)
    pos0 = pos_tk[:, 0]
    pos1 = pos_tk[:, 1]

    w = top_k_weights.astype(jnp.float32)
    w0m = jnp.broadcast_to(w[:, 0:1], (_T, _V))
    w1m = jnp.broadcast_to(w[:, 1:2], (_T, _V))

    x_sorted = _permute_rows(hidden_states, pos0, pos1)
    y_sorted = _ffn(block_expert, x_sorted, gate_up_proj, down_proj)
    return _combine_rows(y_sorted, pos0, pos1, w0m, w1m)


# speedup vs baseline: 2.7539x; 1.2613x over previous
import functools

import jax
import jax.numpy as jnp
from jax import lax
from jax.experimental import pallas as pl
from jax.experimental.pallas import tpu as pltpu
from jax.experimental.pallas import tpu_sc as plsc

_E = 16
_K = 2
_H = 1024
_I = 512
_T = 4096
_B = 128
_P = 10240
_NB = _P // _B
_NC = 2
_NS = 16
_NW = _NC * _NS
_TW = _T // _NW
_GC = 32
_NGC = _TW // _GC
_CC = 16
_NCC = _TW // _CC
_V = 16

_MESH = dict(core_axis_name="c", subcore_axis_name="s")


@functools.partial(
    pl.kernel,
    out_type=jax.ShapeDtypeStruct((_P, _H), jnp.float32),
    mesh=plsc.VectorSubcoreMesh(**_MESH),
    scratch_types=[
        pltpu.VMEM((_GC,), jnp.int32),
        pltpu.VMEM((_GC,), jnp.int32),
        pltpu.VMEM((_GC,), jnp.int32),
        pltpu.VMEM((_GC,), jnp.int32),
        pltpu.VMEM((_GC, _H), jnp.float32),
        pltpu.VMEM((_GC, _H), jnp.float32),
        pltpu.SemaphoreType.DMA,
        pltpu.SemaphoreType.DMA,
    ],
)
def _permute_rows(src, pos0, pos1, out, i0a, i0b, i1a, i1b, bufa, bufb,
                  sem_in, sem_out):
    wid = lax.axis_index("s") * _NC + lax.axis_index("c")
    base = wid * _TW
    i0 = (i0a, i0b)
    i1 = (i1a, i1b)
    buf = (bufa, bufb)
    in_h = [None] * _NGC
    out_h = [None] * _NGC

    def load_idx(c):
        o = base + c * _GC
        pltpu.sync_copy(pos0.at[pl.ds(o, _GC)], i0[c % 2])
        pltpu.sync_copy(pos1.at[pl.ds(o, _GC)], i1[c % 2])

    load_idx(0)
    in_h[0] = pltpu.async_copy(src.at[pl.ds(base, _GC)], buf[0], sem_in)
    for c in range(_NGC):
        in_h[c].wait()
        if c + 1 < _NGC:
            if c >= 1:
                for h in out_h[c - 1]:
                    h.wait()
            load_idx(c + 1)
            o = base + (c + 1) * _GC
            in_h[c + 1] = pltpu.async_copy(
                src.at[pl.ds(o, _GC)], buf[(c + 1) % 2], sem_in)
        out_h[c] = (
            pltpu.async_copy(buf[c % 2], out.at[i0[c % 2]], sem_out),
            pltpu.async_copy(buf[c % 2], out.at[i1[c % 2]], sem_out),
        )
    for c in (_NGC - 2, _NGC - 1):
        for h in out_h[c]:
            h.wait()


@functools.partial(
    pl.kernel,
    out_type=jax.ShapeDtypeStruct((_T, _H), jnp.float32),
    mesh=plsc.VectorSubcoreMesh(**_MESH),
    scratch_types=[
        pltpu.VMEM((_CC,), jnp.int32),
        pltpu.VMEM((_CC,), jnp.int32),
        pltpu.VMEM((_CC,), jnp.int32),
        pltpu.VMEM((_CC,), jnp.int32),
        pltpu.VMEM((_CC, _V), jnp.float32),
        pltpu.VMEM((_CC, _V), jnp.float32),
        pltpu.VMEM((_CC, _V), jnp.float32),
        pltpu.VMEM((_CC, _V), jnp.float32),
        pltpu.VMEM((_CC, _H), jnp.float32),
        pltpu.VMEM((_CC, _H), jnp.float32),
        pltpu.VMEM((_CC, _H), jnp.float32),
        pltpu.VMEM((_CC, _H), jnp.float32),
        pltpu.SemaphoreType.DMA,
        pltpu.SemaphoreType.DMA,
    ],
)
def _combine_rows(y, pos0, pos1, w0m, w1m, out, i0a, i0b, i1a, i1b,
                  w0a, w0b, w1a, w1b, a0, a1, b0, b1, sem_in, sem_out):
    wid = lax.axis_index("s") * _NC + lax.axis_index("c")
    base = wid * _TW
    i0 = (i0a, i0b)
    i1 = (i1a, i1b)
    w0v = (w0a, w0b)
    w1v = (w1a, w1b)
    av = (a0, a1)
    bv = (b0, b1)
    in_h = [None] * _NCC
    out_h = [None] * _NCC

    def start_chunk(c):
        o = base + c * _CC
        pltpu.sync_copy(pos0.at[pl.ds(o, _CC)], i0[c % 2])
        pltpu.sync_copy(pos1.at[pl.ds(o, _CC)], i1[c % 2])
        pltpu.sync_copy(w0m.at[pl.ds(o, _CC)], w0v[c % 2])
        pltpu.sync_copy(w1m.at[pl.ds(o, _CC)], w1v[c % 2])
        in_h[c] = (
            pltpu.async_copy(y.at[i0[c % 2]], av[c % 2], sem_in),
            pltpu.async_copy(y.at[i1[c % 2]], bv[c % 2], sem_in),
        )

    start_chunk(0)
    for c in range(_NCC):
        for h in in_h[c]:
            h.wait()
        if c + 1 < _NCC:
            if c >= 1:
                out_h[c - 1].wait()
            start_chunk(c + 1)
        a_r, b_r = av[c % 2], bv[c % 2]
        w0_r, w1_r = w0v[c % 2], w1v[c % 2]

        def row_body(r, _):
            w0 = w0_r[r, pl.ds(0, _V)]
            w1 = w1_r[r, pl.ds(0, _V)]
            for j in range(_H // _V):
                s = pl.ds(j * _V, _V)
                a_r[r, s] = w0 * a_r[r, s] + w1 * b_r[r, s]
            return 0

        lax.fori_loop(0, _CC, row_body, 0)
        out_h[c] = pltpu.async_copy(
            a_r, out.at[pl.ds(base + c * _CC, _CC)], sem_out)
    out_h[_NCC - 2].wait()
    out_h[_NCC - 1].wait()


def _ffn_block(be_ref, x_ref, gu_ref, dp_ref, y_ref):
    x = x_ref[...]
    gu = gu_ref[0]
    g = lax.dot_general(x, gu, (((1,), (1,)), ((), ())),
                        preferred_element_type=jnp.float32)
    gate = g[:, :_I]
    up = g[:, _I:]
    h = up * (gate * jax.nn.sigmoid(gate))
    dp = dp_ref[0]
    y_ref[...] = lax.dot_general(h, dp, (((1,), (1,)), ((), ())),
                                 preferred_element_type=jnp.float32)


_ffn = pl.pallas_call(
    _ffn_block,
    grid_spec=pltpu.PrefetchScalarGridSpec(
        num_scalar_prefetch=1,
        grid=(_NB,),
        in_specs=[
            pl.BlockSpec((_B, _H), lambda b, be: (b, 0)),
            pl.BlockSpec((1, 2 * _I, _H), lambda b, be: (be[b], 0, 0)),
            pl.BlockSpec((1, _H, _I), lambda b, be: (be[b], 0, 0)),
        ],
        out_specs=pl.BlockSpec((_B, _H), lambda b, be: (b, 0)),
    ),
    out_shape=jax.ShapeDtypeStruct((_P, _H), jnp.float32),
    compiler_params=pltpu.CompilerParams(
        dimension_semantics=("arbitrary",),
    ),
)


def kernel(hidden_states, top_k_index, top_k_weights, gate_up_proj, down_proj):
    tki = top_k_index.astype(jnp.int32)
    eids = jnp.arange(_E, dtype=jnp.int32)
    oh0 = (tki[:, 0:1] == eids[None, :]).astype(jnp.int32)
    oh1 = (tki[:, 1:2] == eids[None, :]).astype(jnp.int32)
    both = oh0 + oh1
    s_incl = jnp.cumsum(both, axis=0)
    s_excl = s_incl - both
    counts = s_incl[-1]
    padded = ((counts + _B - 1) // _B) * _B
    ends = jnp.cumsum(padded)
    offsets = ends - padded
    rank0 = jnp.sum(s_excl * oh0, axis=1)
    rank1 = jnp.sum(s_excl * oh1, axis=1) + (tki[:, 0] == tki[:, 1])
    pos0 = jnp.sum(offsets[None, :] * oh0, axis=1) + rank0
    pos1 = jnp.sum(offsets[None, :] * oh1, axis=1) + rank1
    block_expert = jnp.minimum(
        jnp.sum((jnp.arange(_NB, dtype=jnp.int32)[:, None] * _B >=
                 ends[None, :]).astype(jnp.int32), axis=1),
        _E - 1)

    w = top_k_weights.astype(jnp.float32)
    w0m = jnp.broadcast_to(w[:, 0:1], (_T, _V))
    w1m = jnp.broadcast_to(w[:, 1:2], (_T, _V))

    x_sorted = _permute_rows(hidden_states, pos0, pos1)
    y_sorted = _ffn(block_expert, x_sorted, gate_up_proj, down_proj)
    return _combine_rows(y_sorted, pos0, pos1, w0m, w1m)
